# baseline (speedup 1.0000x reference)
import functools

import jax
import jax.numpy as jnp
from jax import lax
from jax.experimental import pallas as pl
from jax.experimental.pallas import tpu as pltpu
from jax.experimental.pallas import tpu_sc as plsc

_N_SPLITS = 20
_DEFAULT = 1
_ROWS, _COLS = 16384, 26
_NC, _NS = 2, 16
_NW = _NC * _NS
_COLS_W = _ROWS // _NW
_LANES = 16
_CHUNKS = 1
_CW = _COLS_W // _CHUNKS
_CHUNK_ELEMS = _COLS * _CW
_CW_SHIFT = _CW.bit_length() - 1


def _build_sc_kernel():
    mesh = plsc.VectorSubcoreMesh(
        core_axis_name="c", subcore_axis_name="s", num_cores=_NC)

    @functools.partial(
        pl.kernel,
        mesh=mesh,
        out_type=jax.ShapeDtypeStruct((_COLS, _ROWS), jnp.int32),
        scratch_types=(
            [pltpu.VMEM((_COLS, _CW), jnp.int32)] * _CHUNKS
            + [pltpu.SemaphoreType.DMA] * _CHUNKS
            + [pltpu.SemaphoreType.DMA]
        ),
    )
    def sc_lookup(x_hbm, out_hbm, *scratch):
        bufs = scratch[:_CHUNKS]
        in_sems = scratch[_CHUNKS:2 * _CHUNKS]
        out_sem = scratch[2 * _CHUNKS]
        wid = lax.axis_index("s") * _NC + lax.axis_index("c")
        col0 = wid * _COLS_W

        in_copies = [
            pltpu.async_copy(
                x_hbm.at[:, pl.ds(col0 + k * _CW, _CW)], bufs[k], in_sems[k])
            for k in range(_CHUNKS)
        ]
        out_copies = []
        for k in range(_CHUNKS):
            in_copies[k].wait()
            buf = bufs[k]

            @plsc.parallel_loop(0, _CHUNK_ELEMS, step=_LANES, unroll=4)
            def _(i, buf=buf):
                r = i >> _CW_SHIFT
                c = pl.multiple_of(i & (_CW - 1), _LANES)
                x = buf[r, pl.ds(c, _LANES)]
                valid = (x >= -1) & (x <= _N_SPLITS)
                buf[r, pl.ds(c, _LANES)] = jnp.where(
                    valid, x + 1, jnp.int32(_DEFAULT))

            out_copies.append(
                pltpu.async_copy(
                    buf, out_hbm.at[:, pl.ds(col0 + k * _CW, _CW)], out_sem))
        for h in out_copies:
            h.wait()

    return sc_lookup


_sc_lookup = _build_sc_kernel()


@jax.jit
def kernel(inputs):
    return _sc_lookup(inputs.T).T


# speedup vs baseline: 1.0135x; 1.0135x over previous
import functools

import jax
import jax.numpy as jnp
from jax import lax
from jax.experimental import pallas as pl
from jax.experimental.pallas import tpu as pltpu
from jax.experimental.pallas import tpu_sc as plsc

_N_SPLITS = 20
_DEFAULT = 1
_ROWS, _COLS = 16384, 26
_NC, _NS = 2, 16
_NW = _NC * _NS
_COLS_W = _ROWS // _NW
_LANES = 16
_CHUNKS = 2
_CW = _COLS_W // _CHUNKS
_CHUNK_ELEMS = _COLS * _CW
_CW_SHIFT = _CW.bit_length() - 1


def _build_sc_kernel():
    mesh = plsc.VectorSubcoreMesh(
        core_axis_name="c", subcore_axis_name="s", num_cores=_NC)

    @functools.partial(
        pl.kernel,
        mesh=mesh,
        out_type=jax.ShapeDtypeStruct((_COLS, _ROWS), jnp.int32),
        scratch_types=(
            [pltpu.VMEM((_COLS, _CW), jnp.int32)] * _CHUNKS
            + [pltpu.SemaphoreType.DMA] * _CHUNKS
            + [pltpu.SemaphoreType.DMA]
        ),
    )
    def sc_lookup(x_hbm, out_hbm, *scratch):
        bufs = scratch[:_CHUNKS]
        in_sems = scratch[_CHUNKS:2 * _CHUNKS]
        out_sem = scratch[2 * _CHUNKS]
        wid = lax.axis_index("s") * _NC + lax.axis_index("c")
        col0 = wid * _COLS_W

        in_copies = [
            pltpu.async_copy(
                x_hbm.at[:, pl.ds(col0 + k * _CW, _CW)], bufs[k], in_sems[k])
            for k in range(_CHUNKS)
        ]
        out_copies = []
        for k in range(_CHUNKS):
            in_copies[k].wait()
            buf = bufs[k]

            @plsc.parallel_loop(0, _CHUNK_ELEMS, step=_LANES, unroll=4)
            def _(i, buf=buf):
                r = i >> _CW_SHIFT
                c = pl.multiple_of(i & (_CW - 1), _LANES)
                x = buf[r, pl.ds(c, _LANES)]
                valid = (x >= -1) & (x <= _N_SPLITS)
                buf[r, pl.ds(c, _LANES)] = jnp.where(
                    valid, x + 1, jnp.int32(_DEFAULT))

            out_copies.append(
                pltpu.async_copy(
                    buf, out_hbm.at[:, pl.ds(col0 + k * _CW, _CW)], out_sem))
        for h in out_copies:
            h.wait()

    return sc_lookup


_sc_lookup = _build_sc_kernel()


@jax.jit
def kernel(inputs):
    return _sc_lookup(inputs.T).T


# speedup vs baseline: 1.0358x; 1.0219x over previous
import functools

import jax
import jax.numpy as jnp
from jax import lax
from jax.experimental import pallas as pl
from jax.experimental.pallas import tpu as pltpu
from jax.experimental.pallas import tpu_sc as plsc

_N_SPLITS = 20
_DEFAULT = 1
_ROWS, _COLS = 16384, 26
_NC, _NS = 2, 16
_NW = _NC * _NS
_COLS_W = _ROWS // _NW
_LANES = 16
_CHUNKS = 4
_CW = _COLS_W // _CHUNKS
_CHUNK_ELEMS = _COLS * _CW
_CW_SHIFT = _CW.bit_length() - 1


def _build_sc_kernel():
    mesh = plsc.VectorSubcoreMesh(
        core_axis_name="c", subcore_axis_name="s", num_cores=_NC)

    @functools.partial(
        pl.kernel,
        mesh=mesh,
        out_type=jax.ShapeDtypeStruct((_COLS, _ROWS), jnp.int32),
        scratch_types=(
            [pltpu.VMEM((_COLS, _CW), jnp.int32)] * _CHUNKS
            + [pltpu.SemaphoreType.DMA] * _CHUNKS
            + [pltpu.SemaphoreType.DMA]
        ),
    )
    def sc_lookup(x_hbm, out_hbm, *scratch):
        bufs = scratch[:_CHUNKS]
        in_sems = scratch[_CHUNKS:2 * _CHUNKS]
        out_sem = scratch[2 * _CHUNKS]
        wid = lax.axis_index("s") * _NC + lax.axis_index("c")
        col0 = wid * _COLS_W

        in_copies = [
            pltpu.async_copy(
                x_hbm.at[:, pl.ds(col0 + k * _CW, _CW)], bufs[k], in_sems[k])
            for k in range(_CHUNKS)
        ]
        out_copies = []
        for k in range(_CHUNKS):
            in_copies[k].wait()
            buf = bufs[k]

            @plsc.parallel_loop(0, _CHUNK_ELEMS, step=_LANES, unroll=8)
            def _(i, buf=buf):
                r = i >> _CW_SHIFT
                c = pl.multiple_of(i & (_CW - 1), _LANES)
                x = buf[r, pl.ds(c, _LANES)]
                valid = (x >= -1) & (x <= _N_SPLITS)
                buf[r, pl.ds(c, _LANES)] = jnp.where(
                    valid, x + 1, jnp.int32(_DEFAULT))

            out_copies.append(
                pltpu.async_copy(
                    buf, out_hbm.at[:, pl.ds(col0 + k * _CW, _CW)], out_sem))
        for h in out_copies:
            h.wait()

    return sc_lookup


_sc_lookup = _build_sc_kernel()


@jax.jit
def kernel(inputs):
    return _sc_lookup(inputs.T).T


# speedup vs baseline: 1.0370x; 1.0012x over previous
import functools

import jax
import jax.numpy as jnp
from jax import lax
from jax.experimental import pallas as pl
from jax.experimental.pallas import tpu as pltpu
from jax.experimental.pallas import tpu_sc as plsc

_N_SPLITS = 20
_DEFAULT = 1
_ROWS, _COLS = 16384, 26
_NC, _NS = 2, 16
_NW = _NC * _NS
_COLS_W = _ROWS // _NW
_LANES = 16
_CHUNKS = 4
_CW = _COLS_W // _CHUNKS
_CHUNK_ELEMS = _COLS * _CW
_CW_SHIFT = _CW.bit_length() - 1


def _build_sc_kernel():
    mesh = plsc.VectorSubcoreMesh(
        core_axis_name="c", subcore_axis_name="s", num_cores=_NC)

    @functools.partial(
        pl.kernel,
        mesh=mesh,
        out_type=jax.ShapeDtypeStruct((_COLS, _ROWS), jnp.int32),
        scratch_types=(
            [pltpu.VMEM((_COLS, _CW), jnp.int32)] * _CHUNKS
            + [pltpu.SemaphoreType.DMA] * _CHUNKS
            + [pltpu.SemaphoreType.DMA]
        ),
    )
    def sc_lookup(x_hbm, out_hbm, *scratch):
        bufs = scratch[:_CHUNKS]
        in_sems = scratch[_CHUNKS:2 * _CHUNKS]
        out_sem = scratch[2 * _CHUNKS]
        wid = lax.axis_index("s") * _NC + lax.axis_index("c")
        col0 = wid * _COLS_W

        in_copies = [
            pltpu.async_copy(
                x_hbm.at[:, pl.ds(col0 + k * _CW, _CW)], bufs[k], in_sems[k])
            for k in range(_CHUNKS)
        ]
        out_copies = []
        for k in range(_CHUNKS):
            in_copies[k].wait()
            buf = bufs[k]

            @plsc.parallel_loop(0, _CHUNK_ELEMS, step=_LANES, unroll=16)
            def _(i, buf=buf):
                r = i >> _CW_SHIFT
                c = pl.multiple_of(i & (_CW - 1), _LANES)
                x = buf[r, pl.ds(c, _LANES)]
                valid = (x >= -1) & (x <= _N_SPLITS)
                buf[r, pl.ds(c, _LANES)] = jnp.where(
                    valid, x + 1, jnp.int32(_DEFAULT))

            out_copies.append(
                pltpu.async_copy(
                    buf, out_hbm.at[:, pl.ds(col0 + k * _CW, _CW)], out_sem))
        for h in out_copies:
            h.wait()

    return sc_lookup


_sc_lookup = _build_sc_kernel()


@jax.jit
def kernel(inputs):
    return _sc_lookup(inputs.T).T
